# Initial kernel scaffold; baseline (speedup 1.0000x reference)
import jax
import jax.numpy as jnp
from jax.experimental import pallas as pl


def kernel(features, gate_w, gate_b, w1, b1, w2, b2):
    raise NotImplementedError("write your pallas kernel here")


# speedup vs baseline: 11.4261x; 11.4261x over previous
import jax
import jax.numpy as jnp
from jax.experimental import pallas as pl
from jax.experimental.pallas import tpu as pltpu


def _body(f_ref, gw_ref, gb_ref, w1_ref, b1_ref, w2_ref, b2_ref,
          out_ref):
    L, T1, D = f_ref.shape[1], f_ref.shape[2], f_ref.shape[3]
    E = w1_ref.shape[0]

    e = pl.program_id(1)

    cls = f_ref[0, :, 0, :]
    scores = jnp.dot(cls, gw_ref[...],
                     preferred_element_type=jnp.float32) + gb_ref[...]
    m = jnp.max(scores, axis=0, keepdims=True)
    p = jnp.exp(scores - m)
    prob = p / jnp.sum(p, axis=0, keepdims=True)
    ecol = jax.lax.broadcasted_iota(jnp.int32, (L, E), 1) == e
    pe = jnp.sum(jnp.where(ecol, prob, 0.0), axis=1, keepdims=True)

    lidx = jax.lax.broadcasted_iota(jnp.int32, (L, 1), 0)
    v1 = jnp.max(pe)
    i1 = jnp.min(jnp.where(pe >= v1, lidx, L))
    pe2 = jnp.where(lidx == i1, -jnp.inf, pe)
    v2 = jnp.max(pe2)
    i2 = jnp.min(jnp.where(pe2 >= v2, lidx, L))
    t = jnp.exp(v2 - v1)
    wa = 1.0 / (1.0 + t)
    wb = t / (1.0 + t)
    coeff = jnp.where(lidx == i1, wa, 0.0) + jnp.where(lidx == i2, wb, 0.0)

    m257 = jnp.sum(coeff[:, :, None] * f_ref[0], axis=0)
    mixed = m257[1:, :]

    w1e = w1_ref[e]
    h = jnp.dot(mixed, w1e, preferred_element_type=jnp.float32) + b1_ref[e]
    h = jnp.maximum(h, 0.0)
    y_t = jax.lax.dot_general(w2_ref[e], h, (((0,), (1,)), ((), ())),
                              preferred_element_type=jnp.float32)
    y_t = (y_t + b2_ref[e]) * (1.0 / E)

    @pl.when(e == 0)
    def _():
        out_ref[0] = y_t

    @pl.when(e > 0)
    def _():
        out_ref[0] = out_ref[0] + y_t


def _resize_body(avg_ref, r_ref, rt_ref, out_ref):
    C = out_ref.shape[1]
    H = avg_ref.shape[2]
    a1 = jnp.dot(avg_ref[0], rt_ref[...],
                 preferred_element_type=jnp.float32)
    r = r_ref[...]
    for c in range(C):
        out_ref[0, c] = jnp.dot(r, a1[c * H:(c + 1) * H, :],
                                preferred_element_type=jnp.float32)


def kernel(features, gate_w, gate_b, w1, b1, w2, b2):
    B, L, T1, D = features.shape
    E = w1.shape[0]
    C = w2.shape[2]
    IMG = 224
    H = 16

    T = T1 - 1

    r_mat = jax.image.resize(jnp.eye(H, dtype=jnp.float32), (IMG, H),
                             method="bilinear")

    avg = pl.pallas_call(
        _body,
        grid=(B, E),
        in_specs=[
            pl.BlockSpec((1, L, T1, D), lambda b, e: (b, 0, 0, 0)),
            pl.BlockSpec((D, E), lambda b, e: (0, 0)),
            pl.BlockSpec((1, E), lambda b, e: (0, 0)),
            pl.BlockSpec((E, D, D), lambda b, e: (0, 0, 0)),
            pl.BlockSpec((E, 1, D), lambda b, e: (0, 0, 0)),
            pl.BlockSpec((E, D, C), lambda b, e: (0, 0, 0)),
            pl.BlockSpec((E, C, 1), lambda b, e: (0, 0, 0)),
        ],
        out_specs=pl.BlockSpec((1, C, T), lambda b, e: (b, 0, 0)),
        out_shape=jax.ShapeDtypeStruct((B, C, T), jnp.float32),
    )(features, gate_w, gate_b.reshape(1, E), w1, b1.reshape(E, 1, D),
      w2, b2.reshape(E, C, 1))

    avg_m = avg.reshape(B, C * H, H)

    out = pl.pallas_call(
        _resize_body,
        grid=(B,),
        in_specs=[
            pl.BlockSpec((1, C * H, H), lambda b: (b, 0, 0)),
            pl.BlockSpec((IMG, H), lambda b: (0, 0)),
            pl.BlockSpec((H, IMG), lambda b: (0, 0)),
        ],
        out_specs=pl.BlockSpec((1, C, IMG, IMG), lambda b: (b, 0, 0, 0)),
        out_shape=jax.ShapeDtypeStruct((B, C, IMG, IMG), jnp.float32),
    )(avg_m, r_mat, r_mat.T)
    return out


# speedup vs baseline: 12.3359x; 1.0796x over previous
import functools

import jax
import jax.numpy as jnp
from jax import lax
from jax.experimental import pallas as pl
from jax.experimental.pallas import tpu as pltpu
from jax.experimental.pallas import tpu_sc as plsc

_B, _L, _T1, _D = 4, 12, 257, 768
_E, _C, _IMG, _H = 8, 21, 224, 16
_NCORES, _NSUB = 2, 16
_LANES = 16


def _sc_route_body(cls_hbm, gwt_hbm, gb_hbm, out_hbm, cls_v, gw_v, gb_v, out_v):
    wid = lax.axis_index("s") * _NCORES + lax.axis_index("c")
    b = wid // _E
    e = wid % _E

    pltpu.sync_copy(cls_hbm.at[b], cls_v)
    pltpu.sync_copy(gwt_hbm.at[e], gw_v)
    pltpu.sync_copy(gb_hbm, gb_v)

    lane = lax.broadcasted_iota(jnp.int32, (_LANES,), 0)

    scores = jnp.zeros((_LANES,), jnp.float32)
    for l in range(_L):
        def dbody(d, acc, l=l):
            return acc + cls_v[l, pl.ds(d * _LANES, _LANES)] * gw_v[pl.ds(d * _LANES, _LANES)]
        acc = lax.fori_loop(0, _D // _LANES, dbody, jnp.zeros((_LANES,), jnp.float32))
        scores = jnp.where(lane == l, jnp.sum(acc), scores)
    gbe = jnp.sum(jnp.where(lane == e, gb_v[...], 0.0))
    scores = scores + gbe

    masked = jnp.where(lane < _L, scores, -3e38)
    m = jnp.max(masked)
    p = jnp.where(lane < _L, jnp.exp(masked - m), 0.0)
    prob = p / jnp.sum(p)

    v1 = jnp.max(prob)
    i1 = jnp.min(jnp.where(prob >= v1, lane, _LANES))
    prob2 = jnp.where(lane == i1, -1.0, prob)
    v2 = jnp.max(prob2)
    i2 = jnp.min(jnp.where(prob2 >= v2, lane, _LANES))

    t = jnp.exp(jnp.full((_LANES,), v2 - v1, jnp.float32))
    wa = 1.0 / (1.0 + t)
    wb = t * wa

    i1f = i1.astype(jnp.float32)
    i2f = i2.astype(jnp.float32)
    res = jnp.where(lane == 0, wa,
          jnp.where(lane == 1, wb,
          jnp.where(lane == 2, i1f,
          jnp.where(lane == 3, i2f, 0.0))))
    out_v[...] = res
    pltpu.sync_copy(out_v, out_hbm.at[wid])


@functools.cache
def _sc_route():
    return functools.partial(
        pl.kernel,
        out_type=jax.ShapeDtypeStruct((_B * _E, _LANES), jnp.float32),
        mesh=plsc.VectorSubcoreMesh(core_axis_name="c", subcore_axis_name="s",
                                    num_cores=_NCORES),
        compiler_params=pltpu.CompilerParams(needs_layout_passes=False),
        scratch_types=[
            pltpu.VMEM((_L, _D), jnp.float32),
            pltpu.VMEM((_D,), jnp.float32),
            pltpu.VMEM((_LANES,), jnp.float32),
            pltpu.VMEM((_LANES,), jnp.float32),
        ],
    )(_sc_route_body)


def _tc_body(isel_ref, wsel_ref, fa_ref, fb_ref, w1_ref, b1_ref, w2_ref,
             b2_ref, out_ref):
    b = pl.program_id(0)
    e = pl.program_id(1)
    k = b * _E + e

    wa = wsel_ref[k, 0]
    wb = wsel_ref[k, 1]
    mixed = (wa * fa_ref[0, 0] + wb * fb_ref[0, 0])[1:, :]

    w1e = w1_ref[e]
    h = jnp.dot(mixed, w1e, preferred_element_type=jnp.float32) + b1_ref[e]
    h = jnp.maximum(h, 0.0)
    y_t = lax.dot_general(w2_ref[e], h, (((0,), (1,)), ((), ())),
                          preferred_element_type=jnp.float32)
    y_t = (y_t + b2_ref[e]) * (1.0 / _E)

    @pl.when(e == 0)
    def _():
        out_ref[0] = y_t

    @pl.when(e > 0)
    def _():
        out_ref[0] = out_ref[0] + y_t


def _resize_body(avg_ref, r_ref, rt_ref, out_ref):
    a1 = jnp.dot(avg_ref[0], rt_ref[...],
                 preferred_element_type=jnp.float32)
    r = r_ref[...]
    for c in range(_C):
        out_ref[0, c] = jnp.dot(r, a1[c * _H:(c + 1) * _H, :],
                                preferred_element_type=jnp.float32)


def kernel(features, gate_w, gate_b, w1, b1, w2, b2):
    B, L, T1, D = features.shape
    E = w1.shape[0]
    C = w2.shape[2]
    T = T1 - 1

    cls = features[:, :, 0, :]
    gwt = gate_w.T
    gb16 = jnp.pad(gate_b, (0, _LANES - E))
    sel = _sc_route()(cls, gwt, gb16)
    wsel = sel[:, 0:2]
    isel = sel[:, 2:4].astype(jnp.int32)

    grid_spec = pltpu.PrefetchScalarGridSpec(
        num_scalar_prefetch=2,
        grid=(B, E),
        in_specs=[
            pl.BlockSpec((1, 1, T1, D),
                         lambda b, e, i_s, w_s: (b, i_s[b * _E + e, 0], 0, 0)),
            pl.BlockSpec((1, 1, T1, D),
                         lambda b, e, i_s, w_s: (b, i_s[b * _E + e, 1], 0, 0)),
            pl.BlockSpec((E, D, D), lambda b, e, i_s, w_s: (0, 0, 0)),
            pl.BlockSpec((E, 1, D), lambda b, e, i_s, w_s: (0, 0, 0)),
            pl.BlockSpec((E, D, C), lambda b, e, i_s, w_s: (0, 0, 0)),
            pl.BlockSpec((E, C, 1), lambda b, e, i_s, w_s: (0, 0, 0)),
        ],
        out_specs=pl.BlockSpec((1, C, T), lambda b, e, i_s, w_s: (b, 0, 0)),
    )
    avg = pl.pallas_call(
        _tc_body,
        grid_spec=grid_spec,
        out_shape=jax.ShapeDtypeStruct((B, C, T), jnp.float32),
    )(isel, wsel, features, features, w1, b1.reshape(E, 1, D),
      w2, b2.reshape(E, C, 1))

    avg_m = avg.reshape(B, C * _H, _H)

    r_mat = jax.image.resize(jnp.eye(_H, dtype=jnp.float32), (_IMG, _H),
                             method="bilinear")

    out = pl.pallas_call(
        _resize_body,
        grid=(B,),
        in_specs=[
            pl.BlockSpec((1, C * _H, _H), lambda b: (b, 0, 0)),
            pl.BlockSpec((_IMG, _H), lambda b: (0, 0)),
            pl.BlockSpec((_H, _IMG), lambda b: (0, 0)),
        ],
        out_specs=pl.BlockSpec((1, C, _IMG, _IMG), lambda b: (b, 0, 0, 0)),
        out_shape=jax.ShapeDtypeStruct((B, C, _IMG, _IMG), jnp.float32),
    )(avg_m, r_mat, r_mat.T)
    return out
